# baseline (speedup 1.0000x reference)
import functools
import math

import jax
import jax.numpy as jnp
from jax import lax
from jax.experimental import pallas as pl
from jax.experimental.pallas import tpu as pltpu
from jax.experimental.pallas import tpu_sc as plsc

EMB = 64
SCALE = math.sqrt(EMB)
NC = 2
NS = 16
NW = NC * NS
CHUNK = 128
LANES = 16
NBUF = 2


def _sc_embed(tokens_t, table):
    C, R = tokens_t.shape
    n_chunks = C
    mesh = plsc.VectorSubcoreMesh(core_axis_name="c", subcore_axis_name="s")

    @functools.partial(
        pl.kernel,
        out_type=jax.ShapeDtypeStruct((C, EMB, R), jnp.float32),
        mesh=mesh,
        scratch_types=[
            pltpu.VMEM((n_chunks, CHUNK), jnp.int32),
            pltpu.VMEM((NBUF, CHUNK, EMB), jnp.float32),
            pltpu.VMEM((NBUF, EMB, CHUNK), jnp.float32),
            pltpu.SemaphoreType.DMA((NBUF,)),
            pltpu.SemaphoreType.DMA((NBUF,)),
        ],
        compiler_params=pltpu.CompilerParams(
            use_tc_tiling_on_sc=False, needs_layout_passes=False
        ),
    )
    def body(tokens_hbm, table_hbm, out_hbm, idx_v, graw, sbuf, gsem, ssem):
        wid = lax.axis_index("s") * NC + lax.axis_index("c")
        col0 = wid * CHUNK
        pltpu.sync_copy(tokens_hbm.at[:, pl.ds(col0, CHUNK)], idx_v)

        for b in range(NBUF):
            pltpu.async_copy(table_hbm.at[idx_v.at[b]], graw.at[b], gsem.at[b])

        def outer(g, carry):
            for b in range(NBUF):
                j = g * NBUF + b
                pltpu.make_async_copy(
                    table_hbm.at[idx_v.at[b]], graw.at[b], gsem.at[b]
                ).wait()
                @pl.when(j >= NBUF)
                def _():
                    pltpu.make_async_copy(
                        sbuf.at[b],
                        out_hbm.at[0, :, pl.ds(col0, CHUNK)],
                        ssem.at[b],
                    ).wait()

                for grp in range(CHUNK // LANES):
                    rows = lax.iota(jnp.int32, LANES) + grp * LANES

                    def repack(e, carry2):
                        cols = jnp.full((LANES,), e, jnp.int32)
                        v = plsc.load_gather(graw.at[b], [rows, cols])
                        sbuf[b, e, pl.ds(grp * LANES, LANES)] = v * SCALE
                        return carry2

                    lax.fori_loop(0, EMB, repack, 0, unroll=4)

                pltpu.async_copy(
                    sbuf.at[b],
                    out_hbm.at[j, :, pl.ds(col0, CHUNK)],
                    ssem.at[b],
                )

                @pl.when(j + NBUF < n_chunks)
                def _():
                    pltpu.async_copy(
                        table_hbm.at[idx_v.at[j + NBUF]], graw.at[b], gsem.at[b]
                    )
            return carry

        lax.fori_loop(0, n_chunks // NBUF, outer, 0)

        for b in range(NBUF):
            pltpu.make_async_copy(
                sbuf.at[b], out_hbm.at[0, :, pl.ds(col0, CHUNK)], ssem.at[b]
            ).wait()

    return body(tokens_t, table)


def kernel(tokens, table):
    R, C = tokens.shape
    out_t = _sc_embed(tokens.T, table)
    return jnp.transpose(out_t, (2, 0, 1))


# speedup vs baseline: 1.0468x; 1.0468x over previous
import functools
import math

import jax
import jax.numpy as jnp
from jax import lax
from jax.experimental import pallas as pl
from jax.experimental.pallas import tpu as pltpu
from jax.experimental.pallas import tpu_sc as plsc

EMB = 64
PAIR = 128
SCALE = math.sqrt(EMB)
NC = 2
NS = 16
NW = NC * NS
CHUNK = 128
LANES = 16
GRPS = CHUNK // LANES
NBUF = 2
SBUF = 2
PREP_BLK = 512
HALF_V = 512 * 977


def _tc_prep(table_t):
    def body(lo_ref, hi_ref, out_ref):
        out_ref[:, 0:EMB] = jnp.swapaxes(lo_ref[:, :], 0, 1)
        out_ref[:, EMB:PAIR] = jnp.swapaxes(hi_ref[:, :], 0, 1)

    nblk = HALF_V // PREP_BLK
    return pl.pallas_call(
        body,
        grid=(nblk,),
        in_specs=[
            pl.BlockSpec((EMB, PREP_BLK), lambda i: (0, i)),
            pl.BlockSpec((EMB, PREP_BLK), lambda i: (0, i + nblk)),
        ],
        out_specs=pl.BlockSpec((PREP_BLK, PAIR), lambda i: (i, 0)),
        out_shape=jax.ShapeDtypeStruct((HALF_V, PAIR), jnp.float32),
    )(table_t, table_t)


def _sc_embed(tokens_t, table_r):
    C, R = tokens_t.shape
    n_chunks = C
    mesh = plsc.VectorSubcoreMesh(core_axis_name="c", subcore_axis_name="s")

    @functools.partial(
        pl.kernel,
        out_type=jax.ShapeDtypeStruct((C, EMB, R), jnp.float32),
        mesh=mesh,
        scratch_types=[
            pltpu.VMEM((n_chunks, CHUNK), jnp.int32),
            pltpu.VMEM((n_chunks, CHUNK), jnp.int32),
            pltpu.VMEM((NBUF, CHUNK, PAIR), jnp.float32),
            pltpu.VMEM((SBUF, EMB, CHUNK), jnp.float32),
            pltpu.SemaphoreType.DMA((NBUF,)),
            pltpu.SemaphoreType.DMA((SBUF,)),
        ],
        compiler_params=pltpu.CompilerParams(
            use_tc_tiling_on_sc=True, needs_layout_passes=False
        ),
    )
    def body(tokens_hbm, table_hbm, out_hbm, idx_v, par_v, graw, sbuf,
             gsem, ssem):
        wid = lax.axis_index("s") * NC + lax.axis_index("c")
        col0 = wid * CHUNK
        pltpu.sync_copy(tokens_hbm.at[:, pl.ds(col0, CHUNK)], idx_v)

        def split_row(r, carry):
            for g in range(GRPS):
                sl = pl.ds(g * LANES, LANES)
                v = idx_v[r, sl]
                hi = (v >= HALF_V).astype(jnp.int32)
                par_v[r, sl] = hi << 6
                idx_v[r, sl] = v - hi * HALF_V
            return carry

        lax.fori_loop(0, n_chunks, split_row, 0, unroll=4)

        for b in range(NBUF):
            pltpu.async_copy(table_hbm.at[idx_v.at[b]], graw.at[b], gsem.at[b])

        def outer(gi, carry):
            for b in range(NBUF):
                j = gi * NBUF + b
                s = b % SBUF
                pltpu.make_async_copy(
                    table_hbm.at[idx_v.at[b]], graw.at[b], gsem.at[b]
                ).wait()
                @pl.when(j >= SBUF)
                def _():
                    pltpu.make_async_copy(
                        sbuf.at[s],
                        out_hbm.at[0, :, pl.ds(col0, CHUNK)],
                        ssem.at[s],
                    ).wait()

                for grp in range(GRPS):
                    rows = lax.iota(jnp.int32, LANES) + grp * LANES
                    par = par_v[j, pl.ds(grp * LANES, LANES)]

                    def repack(e, carry2):
                        cols = jnp.full((LANES,), e, jnp.int32) + par
                        v = plsc.load_gather(graw.at[b], [rows, cols])
                        sbuf[s, e, pl.ds(grp * LANES, LANES)] = v * SCALE
                        return carry2

                    lax.fori_loop(0, EMB, repack, 0, unroll=4)

                pltpu.async_copy(
                    sbuf.at[s],
                    out_hbm.at[j, :, pl.ds(col0, CHUNK)],
                    ssem.at[s],
                )

                @pl.when(j + NBUF < n_chunks)
                def _():
                    pltpu.async_copy(
                        table_hbm.at[idx_v.at[j + NBUF]], graw.at[b], gsem.at[b]
                    )
            return carry

        lax.fori_loop(0, n_chunks // NBUF, outer, 0)

        for s in range(SBUF):
            pltpu.make_async_copy(
                sbuf.at[s], out_hbm.at[0, :, pl.ds(col0, CHUNK)], ssem.at[s]
            ).wait()

    return body(tokens_t, table_r)


def kernel(tokens, table):
    table_r = _tc_prep(table.T)
    out_t = _sc_embed(tokens.T, table_r)
    return jnp.transpose(out_t, (2, 0, 1))
